# baseline (speedup 1.0000x reference)
import functools

import jax
import jax.numpy as jnp
from jax import lax
from jax.experimental import pallas as pl
from jax.experimental.pallas import tpu as pltpu
from jax.experimental.pallas import tpu_sc as plsc

BN_EPS = 1e-5


@functools.partial(jax.jit, static_argnames=("n", "e", "d"))
def _sc_aggregate(x, src, dst, attr, emb, *, n, e, d):
    info = plsc.get_sparse_core_info()
    nc, ns, nl = info.num_cores, info.num_subcores, info.num_lanes
    nw = nc * ns
    ew = e // nw
    S = 80
    ng = ew // S
    nemb = emb.shape[0]
    assert e % nw == 0 and ew % S == 0 and S % 8 == 0
    ZR = (n // ns) // 8 * 8
    tail = n - ns * ZR
    assert tail % 8 == 0 and tail <= ZR

    mesh = plsc.VectorSubcoreMesh(core_axis_name="c", subcore_axis_name="s")

    @functools.partial(
        pl.kernel,
        out_type=jax.ShapeDtypeStruct((nc, n, d), jnp.float32),
        mesh=mesh,
        scratch_types=[
            pltpu.VMEM((3, 1, S), jnp.int32),
            pltpu.VMEM((3, 1, S), jnp.int32),
            pltpu.VMEM((3, 1, S), jnp.int32),
            pltpu.VMEM((2, S, d), jnp.float32),
            pltpu.VMEM((nemb, d), jnp.float32),
            pltpu.VMEM_SHARED((n, d), jnp.float32),
            pltpu.SemaphoreType.DMA,
            pltpu.SemaphoreType.DMA,
            pltpu.SemaphoreType.DMA,
            pltpu.SemaphoreType.DMA,
        ],
    )
    def k(x_hbm, src_hbm, dst_hbm, attr_hbm, emb_hbm, out_hbm,
          src_v, dst_v, attr_v, rows_v, emb_tab, aggr_sh,
          sem_g, sem_e, sem_i, sem_s):
        cid = lax.axis_index("c")
        sid = lax.axis_index("s")
        wid = sid * nc + cid
        ebase = wid * ew

        pltpu.sync_copy(emb_hbm, emb_tab)

        def zrow(i, carry):
            for j in range(d // nl):
                rows_v[0, i, pl.ds(j * nl, nl)] = jnp.zeros((nl,), jnp.float32)
            return carry
        lax.fori_loop(0, S, zrow, 0)
        zero_v = rows_v.at[0, pl.ds(0, 80)]
        zbase = sid * ZR
        for t in range(ZR // 80):
            pltpu.sync_copy(zero_v, aggr_sh.at[pl.ds(zbase + t * 80, 80)])
        zrem = ZR % 80
        if zrem:
            pltpu.sync_copy(rows_v.at[0, pl.ds(0, zrem)],
                            aggr_sh.at[pl.ds(zbase + ZR - zrem, zrem)])

        @pl.when(sid == ns - 1)
        def _():
            pltpu.sync_copy(rows_v.at[0, pl.ds(0, tail)],
                            aggr_sh.at[pl.ds(ns * ZR, tail)])
        plsc.subcore_barrier()

        for b in range(2):
            off = ebase + b * S
            pltpu.sync_copy(src_hbm.at[pl.ds(off, S)], src_v.at[b, 0])
            pltpu.sync_copy(dst_hbm.at[pl.ds(off, S)], dst_v.at[b, 0])
            pltpu.sync_copy(attr_hbm.at[pl.ds(off, S)], attr_v.at[b, 0])
        pltpu.async_copy(x_hbm.at[src_v.at[0, 0]], rows_v.at[0], sem_g).wait()

        def group(g, carry):
            rb = lax.rem(g, 2)
            rb1 = lax.rem(g + 1, 2)
            b3 = lax.rem(g, 3)
            b31 = lax.rem(g + 1, 3)
            b32 = lax.rem(g + 2, 3)

            @pl.when(g > 0)
            def _():
                pltpu.make_async_copy(rows_v.at[rb1],
                                      aggr_sh.at[dst_v.at[b32, 0]],
                                      sem_s).wait()

            @pl.when(g < ng - 1)
            def _():
                pltpu.async_copy(x_hbm.at[src_v.at[b31, 0]],
                                 rows_v.at[rb1], sem_g)

            @pl.when(g < ng - 2)
            def _():
                off2 = ebase + (g + 2) * S
                pltpu.async_copy(src_hbm.at[pl.ds(off2, S)],
                                 src_v.at[b32, 0], sem_i)
                pltpu.async_copy(dst_hbm.at[pl.ds(off2, S)],
                                 dst_v.at[b32, 0], sem_i)
                pltpu.async_copy(attr_hbm.at[pl.ds(off2, S)],
                                 attr_v.at[b32, 0], sem_i)

            @plsc.parallel_loop(0, S, nl, unroll=2)
            def body(i0):
                avec = attr_v[b3, 0, pl.ds(i0, nl)]
                for lane in range(nl):
                    ai = avec[lane]
                    for j in range(d // nl):
                        s = pl.ds(j * nl, nl)
                        v = rows_v[rb, i0 + lane, s] + emb_tab[ai, s]
                        rows_v[rb, i0 + lane, s] = jnp.maximum(v, 0.0)

            pltpu.async_copy(rows_v.at[rb], aggr_sh.at[dst_v.at[b3, 0]],
                             sem_s, add=True)

            @pl.when(g < ng - 1)
            def _():
                pltpu.make_async_copy(x_hbm.at[src_v.at[b31, 0]],
                                      rows_v.at[rb1], sem_g).wait()

            @pl.when(g < ng - 2)
            def _():
                off2 = ebase + (g + 2) * S
                pltpu.make_async_copy(src_hbm.at[pl.ds(off2, S)],
                                      src_v.at[b32, 0], sem_i).wait()
                pltpu.make_async_copy(dst_hbm.at[pl.ds(off2, S)],
                                      dst_v.at[b32, 0], sem_i).wait()
                pltpu.make_async_copy(attr_hbm.at[pl.ds(off2, S)],
                                      attr_v.at[b32, 0], sem_i).wait()
            return carry
        lax.fori_loop(0, ng, group, 0)
        pltpu.make_async_copy(rows_v.at[(ng - 1) % 2],
                              aggr_sh.at[dst_v.at[(ng - 1) % 3, 0]],
                              sem_s).wait()
        plsc.subcore_barrier()

        pltpu.sync_copy(aggr_sh.at[pl.ds(sid * ZR, ZR)],
                        out_hbm.at[cid, pl.ds(sid * ZR, ZR)])

        @pl.when(sid == ns - 1)
        def _():
            pltpu.sync_copy(aggr_sh.at[pl.ds(ns * ZR, tail)],
                            out_hbm.at[cid, pl.ds(ns * ZR, tail)])

    return k(x, src, dst, attr, emb)


def _tc_h1(x, parts, eps_arr, W1, b1, *, n, d, blk):
    nb = n // blk

    def ka(eps_ref, x_ref, parts_ref, w1_ref, b1_ref, h1_ref, sums_ref):
        i = pl.program_id(0)
        z = ((1.0 + eps_ref[0]) * x_ref[...]
             + parts_ref[0] + parts_ref[1])
        h = jnp.dot(z, w1_ref[...], preferred_element_type=jnp.float32)
        h = h + b1_ref[...]
        h1_ref[...] = h

        @pl.when(i == 0)
        def _():
            sums_ref[...] = jnp.zeros_like(sums_ref)
        sums_ref[0:1, :] += jnp.sum(h, axis=0, keepdims=True)
        sums_ref[1:2, :] += jnp.sum(h * h, axis=0, keepdims=True)

    return pl.pallas_call(
        ka,
        grid=(nb,),
        in_specs=[
            pl.BlockSpec(memory_space=pltpu.SMEM),
            pl.BlockSpec((blk, d), lambda i: (i, 0)),
            pl.BlockSpec((2, blk, d), lambda i: (0, i, 0)),
            pl.BlockSpec((d, d), lambda i: (0, 0)),
            pl.BlockSpec((1, d), lambda i: (0, 0)),
        ],
        out_specs=[
            pl.BlockSpec((blk, d), lambda i: (i, 0)),
            pl.BlockSpec((8, d), lambda i: (0, 0)),
        ],
        out_shape=[
            jax.ShapeDtypeStruct((n, d), jnp.float32),
            jax.ShapeDtypeStruct((8, d), jnp.float32),
        ],
    )(eps_arr, x, parts, W1, b1)


def _tc_out(h1, sums, gamma, beta, W2, b2, *, n, d, blk):
    nb = n // blk
    inv_n = 1.0 / n

    def kb(h1_ref, sums_ref, g_ref, be_ref, w2_ref, b2_ref, o_ref):
        mu = sums_ref[0:1, :] * inv_n
        var = sums_ref[1:2, :] * inv_n - mu * mu
        a = g_ref[...] * lax.rsqrt(var + BN_EPS)
        c = be_ref[...] - mu * a
        h = jnp.maximum(h1_ref[...] * a + c, 0.0)
        o_ref[...] = (jnp.dot(h, w2_ref[...], preferred_element_type=jnp.float32)
                      + b2_ref[...])

    return pl.pallas_call(
        kb,
        grid=(nb,),
        in_specs=[
            pl.BlockSpec((blk, d), lambda i: (i, 0)),
            pl.BlockSpec((8, d), lambda i: (0, 0)),
            pl.BlockSpec((1, d), lambda i: (0, 0)),
            pl.BlockSpec((1, d), lambda i: (0, 0)),
            pl.BlockSpec((d, d), lambda i: (0, 0)),
            pl.BlockSpec((1, d), lambda i: (0, 0)),
        ],
        out_specs=pl.BlockSpec((blk, d), lambda i: (i, 0)),
        out_shape=jax.ShapeDtypeStruct((n, d), jnp.float32),
    )(h1, sums, gamma, beta, W2, b2)


def kernel(x, edge_index, edge_attr, edge_emb_w, eps, W1, b1, gamma, beta,
           W2, b2):
    n, d = x.shape
    e = edge_attr.shape[0]
    src = edge_index[0]
    dst = edge_index[1]

    parts = _sc_aggregate(x, src, dst, edge_attr, edge_emb_w, n=n, e=e, d=d)

    blk = 2000
    eps_arr = jnp.reshape(eps, (1,)).astype(jnp.float32)
    h1, sums = _tc_h1(x, parts, eps_arr, W1, jnp.reshape(b1, (1, d)),
                      n=n, d=d, blk=blk)
    out = _tc_out(h1, sums, jnp.reshape(gamma, (1, d)),
                  jnp.reshape(beta, (1, d)), W2, jnp.reshape(b2, (1, d)),
                  n=n, d=d, blk=blk)
    return out


# speedup vs baseline: 2.6572x; 2.6572x over previous
import functools

import jax
import jax.numpy as jnp
from jax import lax
from jax.experimental import pallas as pl
from jax.experimental.pallas import tpu as pltpu
from jax.experimental.pallas import tpu_sc as plsc

BN_EPS = 1e-5


@functools.partial(jax.jit, static_argnames=("n", "e", "d"))
def _sc_aggregate(x, src, dst, attr, emb, *, n, e, d):
    info = plsc.get_sparse_core_info()
    nc, ns, nl = info.num_cores, info.num_subcores, info.num_lanes
    nw = nc * ns
    ew = e // nw
    S = 80
    ng = ew // S
    nemb = emb.shape[0]
    assert e % nw == 0 and ew % S == 0 and S % 8 == 0
    ZR = (n // ns) // 8 * 8
    tail = n - ns * ZR
    assert tail % 8 == 0 and tail <= ZR

    mesh = plsc.VectorSubcoreMesh(core_axis_name="c", subcore_axis_name="s")

    @functools.partial(
        pl.kernel,
        out_type=jax.ShapeDtypeStruct((nc, n, d), jnp.float32),
        mesh=mesh,
        scratch_types=[
            pltpu.VMEM((3, 1, S), jnp.int32),
            pltpu.VMEM((3, 1, S), jnp.int32),
            pltpu.VMEM((3, 1, S), jnp.int32),
            pltpu.VMEM((2, S, d), jnp.float32),
            pltpu.VMEM((2, S, d), jnp.float32),
            pltpu.VMEM_SHARED((nemb, d), jnp.float32),
            pltpu.VMEM_SHARED((n, d), jnp.float32),
            pltpu.SemaphoreType.DMA,
            pltpu.SemaphoreType.DMA,
            pltpu.SemaphoreType.DMA,
            pltpu.SemaphoreType.DMA,
        ],
    )
    def k(x_hbm, src_hbm, dst_hbm, attr_hbm, emb_hbm, out_hbm,
          src_v, dst_v, attr_v, rows_v, erows_v, emb_sh, aggr_sh,
          sem_g, sem_e, sem_i, sem_s):
        cid = lax.axis_index("c")
        sid = lax.axis_index("s")
        wid = sid * nc + cid
        ebase = wid * ew

        @pl.when(sid == 0)
        def _():
            pltpu.sync_copy(emb_hbm, emb_sh)

        def zrow(i, carry):
            for j in range(d // nl):
                rows_v[0, i, pl.ds(j * nl, nl)] = jnp.zeros((nl,), jnp.float32)
            return carry
        lax.fori_loop(0, S, zrow, 0)
        zero_v = rows_v.at[0, pl.ds(0, 80)]
        zbase = sid * ZR
        for t in range(ZR // 80):
            pltpu.sync_copy(zero_v, aggr_sh.at[pl.ds(zbase + t * 80, 80)])
        zrem = ZR % 80
        if zrem:
            pltpu.sync_copy(rows_v.at[0, pl.ds(0, zrem)],
                            aggr_sh.at[pl.ds(zbase + ZR - zrem, zrem)])

        @pl.when(sid == ns - 1)
        def _():
            pltpu.sync_copy(rows_v.at[0, pl.ds(0, tail)],
                            aggr_sh.at[pl.ds(ns * ZR, tail)])
        plsc.subcore_barrier()

        for b in range(2):
            off = ebase + b * S
            pltpu.sync_copy(src_hbm.at[pl.ds(off, S)], src_v.at[b, 0])
            pltpu.sync_copy(dst_hbm.at[pl.ds(off, S)], dst_v.at[b, 0])
            pltpu.sync_copy(attr_hbm.at[pl.ds(off, S)], attr_v.at[b, 0])
        cg = pltpu.async_copy(x_hbm.at[src_v.at[0, 0]], rows_v.at[0], sem_g)
        ce = pltpu.async_copy(emb_sh.at[attr_v.at[0, 0]], erows_v.at[0], sem_e)
        cg.wait()
        ce.wait()

        def group(g, carry):
            rb = lax.rem(g, 2)
            rb1 = lax.rem(g + 1, 2)
            b3 = lax.rem(g, 3)
            b31 = lax.rem(g + 1, 3)
            b32 = lax.rem(g + 2, 3)

            @pl.when(g > 0)
            def _():
                pltpu.make_async_copy(rows_v.at[rb1],
                                      aggr_sh.at[dst_v.at[b32, 0]],
                                      sem_s).wait()

            @pl.when(g < ng - 1)
            def _():
                pltpu.async_copy(x_hbm.at[src_v.at[b31, 0]],
                                 rows_v.at[rb1], sem_g)
                pltpu.async_copy(emb_sh.at[attr_v.at[b31, 0]],
                                 erows_v.at[rb1], sem_e)

            @pl.when(g < ng - 2)
            def _():
                off2 = ebase + (g + 2) * S
                pltpu.async_copy(src_hbm.at[pl.ds(off2, S)],
                                 src_v.at[b32, 0], sem_i)
                pltpu.async_copy(dst_hbm.at[pl.ds(off2, S)],
                                 dst_v.at[b32, 0], sem_i)
                pltpu.async_copy(attr_hbm.at[pl.ds(off2, S)],
                                 attr_v.at[b32, 0], sem_i)

            @plsc.parallel_loop(0, S, 1, unroll=4)
            def body(i):
                for j in range(d // nl):
                    s = pl.ds(j * nl, nl)
                    v = rows_v[rb, i, s] + erows_v[rb, i, s]
                    rows_v[rb, i, s] = jnp.maximum(v, 0.0)

            pltpu.async_copy(rows_v.at[rb], aggr_sh.at[dst_v.at[b3, 0]],
                             sem_s, add=True)

            @pl.when(g < ng - 1)
            def _():
                pltpu.make_async_copy(x_hbm.at[src_v.at[b31, 0]],
                                      rows_v.at[rb1], sem_g).wait()
                pltpu.make_async_copy(emb_sh.at[attr_v.at[b31, 0]],
                                      erows_v.at[rb1], sem_e).wait()

            @pl.when(g < ng - 2)
            def _():
                off2 = ebase + (g + 2) * S
                pltpu.make_async_copy(src_hbm.at[pl.ds(off2, S)],
                                      src_v.at[b32, 0], sem_i).wait()
                pltpu.make_async_copy(dst_hbm.at[pl.ds(off2, S)],
                                      dst_v.at[b32, 0], sem_i).wait()
                pltpu.make_async_copy(attr_hbm.at[pl.ds(off2, S)],
                                      attr_v.at[b32, 0], sem_i).wait()
            return carry
        lax.fori_loop(0, ng, group, 0)
        pltpu.make_async_copy(rows_v.at[(ng - 1) % 2],
                              aggr_sh.at[dst_v.at[(ng - 1) % 3, 0]],
                              sem_s).wait()
        plsc.subcore_barrier()

        pltpu.sync_copy(aggr_sh.at[pl.ds(sid * ZR, ZR)],
                        out_hbm.at[cid, pl.ds(sid * ZR, ZR)])

        @pl.when(sid == ns - 1)
        def _():
            pltpu.sync_copy(aggr_sh.at[pl.ds(ns * ZR, tail)],
                            out_hbm.at[cid, pl.ds(ns * ZR, tail)])

    return k(x, src, dst, attr, emb)


def _tc_mlp(x, parts, eps_arr, W1, b1, gamma, beta, W2, b2, *, n, d, blk):
    nb = n // blk
    inv_n = 1.0 / n

    def kf(eps_ref, x_ref, parts_ref, w1_ref, b1_ref, g_ref, be_ref,
           w2_ref, b2_ref, o_ref, h1_scr, sums_scr):
        p = pl.program_id(0)
        i = pl.program_id(1)

        @pl.when(p == 0)
        def _():
            z = ((1.0 + eps_ref[0]) * x_ref[...]
                 + parts_ref[0] + parts_ref[1])
            h = jnp.dot(z, w1_ref[...], preferred_element_type=jnp.float32)
            h = h + b1_ref[...]
            h1_scr[pl.ds(i * blk, blk), :] = h

            @pl.when(i == 0)
            def _():
                sums_scr[...] = jnp.zeros_like(sums_scr)
            sums_scr[0:1, :] += jnp.sum(h, axis=0, keepdims=True)
            sums_scr[1:2, :] += jnp.sum(h * h, axis=0, keepdims=True)

        @pl.when(p == 1)
        def _():
            mu = sums_scr[0:1, :] * inv_n
            var = sums_scr[1:2, :] * inv_n - mu * mu
            a = g_ref[...] * lax.rsqrt(var + BN_EPS)
            c = be_ref[...] - mu * a
            h = jnp.maximum(h1_scr[pl.ds(i * blk, blk), :] * a + c, 0.0)
            o_ref[...] = (jnp.dot(h, w2_ref[...],
                                  preferred_element_type=jnp.float32)
                          + b2_ref[...])

    return pl.pallas_call(
        kf,
        grid=(2, nb),
        in_specs=[
            pl.BlockSpec(memory_space=pltpu.SMEM),
            pl.BlockSpec((blk, d), lambda p, i: ((1 - p) * i, 0)),
            pl.BlockSpec((2, blk, d), lambda p, i: (0, (1 - p) * i, 0)),
            pl.BlockSpec((d, d), lambda p, i: (0, 0)),
            pl.BlockSpec((1, d), lambda p, i: (0, 0)),
            pl.BlockSpec((1, d), lambda p, i: (0, 0)),
            pl.BlockSpec((1, d), lambda p, i: (0, 0)),
            pl.BlockSpec((d, d), lambda p, i: (0, 0)),
            pl.BlockSpec((1, d), lambda p, i: (0, 0)),
        ],
        out_specs=pl.BlockSpec((blk, d), lambda p, i: (i, 0)),
        out_shape=jax.ShapeDtypeStruct((n, d), jnp.float32),
        scratch_shapes=[
            pltpu.VMEM((n, d), jnp.float32),
            pltpu.VMEM((8, d), jnp.float32),
        ],
    )(eps_arr, x, parts, W1, b1, gamma, beta, W2, b2)


def kernel(x, edge_index, edge_attr, edge_emb_w, eps, W1, b1, gamma, beta,
           W2, b2):
    n, d = x.shape
    e = edge_attr.shape[0]
    src = edge_index[0]
    dst = edge_index[1]

    parts = _sc_aggregate(x, src, dst, edge_attr, edge_emb_w, n=n, e=e, d=d)

    blk = 2000
    eps_arr = jnp.reshape(eps, (1,)).astype(jnp.float32)
    out = _tc_mlp(x, parts, eps_arr, W1, jnp.reshape(b1, (1, d)),
                  jnp.reshape(gamma, (1, d)), jnp.reshape(beta, (1, d)),
                  W2, jnp.reshape(b2, (1, d)), n=n, d=d, blk=blk)
    return out


# speedup vs baseline: 2.6908x; 1.0126x over previous
import functools

import jax
import jax.numpy as jnp
from jax import lax
from jax.experimental import pallas as pl
from jax.experimental.pallas import tpu as pltpu
from jax.experimental.pallas import tpu_sc as plsc

BN_EPS = 1e-5


@functools.partial(jax.jit, static_argnames=("n", "e", "d"))
def _sc_aggregate(x, src, dst, attr, emb, *, n, e, d):
    info = plsc.get_sparse_core_info()
    nc, ns, nl = info.num_cores, info.num_subcores, info.num_lanes
    nw = nc * ns
    ew = e // nw
    S = 80
    ng = ew // S
    nemb = emb.shape[0]
    assert e % nw == 0 and ew % S == 0 and S % 8 == 0
    ZR = (n // ns) // 8 * 8
    tail = n - ns * ZR
    assert tail % 8 == 0 and tail <= ZR

    mesh = plsc.VectorSubcoreMesh(core_axis_name="c", subcore_axis_name="s")

    @functools.partial(
        pl.kernel,
        out_type=jax.ShapeDtypeStruct((nc, n, d), jnp.float32),
        mesh=mesh,
        scratch_types=[
            pltpu.VMEM((3, 1, S), jnp.int32),
            pltpu.VMEM((3, 1, S), jnp.int32),
            pltpu.VMEM((3, 1, S), jnp.int32),
            pltpu.VMEM((2, S, d), jnp.float32),
            pltpu.VMEM((2, S, d), jnp.float32),
            pltpu.VMEM_SHARED((nemb, d), jnp.float32),
            pltpu.VMEM_SHARED((n, d), jnp.float32),
            pltpu.SemaphoreType.DMA,
            pltpu.SemaphoreType.DMA,
            pltpu.SemaphoreType.DMA,
            pltpu.SemaphoreType.DMA,
        ],
    )
    def k(x_hbm, src_hbm, dst_hbm, attr_hbm, emb_hbm, out_hbm,
          src_v, dst_v, attr_v, rows_v, erows_v, emb_sh, aggr_sh,
          sem_g, sem_e, sem_i, sem_s):
        cid = lax.axis_index("c")
        sid = lax.axis_index("s")
        wid = sid * nc + cid
        ebase = wid * ew

        @pl.when(sid == 0)
        def _():
            pltpu.sync_copy(emb_hbm, emb_sh)

        def zrow(i, carry):
            for j in range(d // nl):
                rows_v[0, i, pl.ds(j * nl, nl)] = jnp.zeros((nl,), jnp.float32)
            return carry
        lax.fori_loop(0, S, zrow, 0)
        zero_v = rows_v.at[0, pl.ds(0, 80)]
        zbase = sid * ZR
        for t in range(ZR // 80):
            pltpu.sync_copy(zero_v, aggr_sh.at[pl.ds(zbase + t * 80, 80)])
        zrem = ZR % 80
        if zrem:
            pltpu.sync_copy(rows_v.at[0, pl.ds(0, zrem)],
                            aggr_sh.at[pl.ds(zbase + ZR - zrem, zrem)])

        @pl.when(sid == ns - 1)
        def _():
            pltpu.sync_copy(rows_v.at[0, pl.ds(0, tail)],
                            aggr_sh.at[pl.ds(ns * ZR, tail)])
        plsc.subcore_barrier()

        for b in range(2):
            off = ebase + b * S
            pltpu.sync_copy(src_hbm.at[pl.ds(off, S)], src_v.at[b, 0])
            pltpu.sync_copy(dst_hbm.at[pl.ds(off, S)], dst_v.at[b, 0])
            pltpu.sync_copy(attr_hbm.at[pl.ds(off, S)], attr_v.at[b, 0])
        cg = pltpu.async_copy(x_hbm.at[src_v.at[0, 0]], rows_v.at[0], sem_g)
        ce = pltpu.async_copy(emb_sh.at[attr_v.at[0, 0]], erows_v.at[0], sem_e)
        cg.wait()
        ce.wait()

        def group(g, carry):
            rb = lax.rem(g, 2)
            rb1 = lax.rem(g + 1, 2)
            b3 = lax.rem(g, 3)
            b31 = lax.rem(g + 1, 3)
            b32 = lax.rem(g + 2, 3)

            @pl.when(g > 0)
            def _():
                pltpu.make_async_copy(rows_v.at[rb1],
                                      aggr_sh.at[dst_v.at[b32, 0]],
                                      sem_s).wait()

            @pl.when(g < ng - 1)
            def _():
                pltpu.async_copy(x_hbm.at[src_v.at[b31, 0]],
                                 rows_v.at[rb1], sem_g)
                pltpu.async_copy(emb_sh.at[attr_v.at[b31, 0]],
                                 erows_v.at[rb1], sem_e)

            @pl.when(g < ng - 2)
            def _():
                off2 = ebase + (g + 2) * S
                pltpu.async_copy(src_hbm.at[pl.ds(off2, S)],
                                 src_v.at[b32, 0], sem_i)
                pltpu.async_copy(dst_hbm.at[pl.ds(off2, S)],
                                 dst_v.at[b32, 0], sem_i)
                pltpu.async_copy(attr_hbm.at[pl.ds(off2, S)],
                                 attr_v.at[b32, 0], sem_i)

            @plsc.parallel_loop(0, S, 1, unroll=8)
            def body(i):
                for j in range(d // nl):
                    s = pl.ds(j * nl, nl)
                    v = rows_v[rb, i, s] + erows_v[rb, i, s]
                    rows_v[rb, i, s] = jnp.maximum(v, 0.0)

            pltpu.async_copy(rows_v.at[rb], aggr_sh.at[dst_v.at[b3, 0]],
                             sem_s, add=True)

            @pl.when(g < ng - 1)
            def _():
                pltpu.make_async_copy(x_hbm.at[src_v.at[b31, 0]],
                                      rows_v.at[rb1], sem_g).wait()
                pltpu.make_async_copy(emb_sh.at[attr_v.at[b31, 0]],
                                      erows_v.at[rb1], sem_e).wait()

            @pl.when(g < ng - 2)
            def _():
                off2 = ebase + (g + 2) * S
                pltpu.make_async_copy(src_hbm.at[pl.ds(off2, S)],
                                      src_v.at[b32, 0], sem_i).wait()
                pltpu.make_async_copy(dst_hbm.at[pl.ds(off2, S)],
                                      dst_v.at[b32, 0], sem_i).wait()
                pltpu.make_async_copy(attr_hbm.at[pl.ds(off2, S)],
                                      attr_v.at[b32, 0], sem_i).wait()
            return carry
        lax.fori_loop(0, ng, group, 0)
        pltpu.make_async_copy(rows_v.at[(ng - 1) % 2],
                              aggr_sh.at[dst_v.at[(ng - 1) % 3, 0]],
                              sem_s).wait()
        plsc.subcore_barrier()

        pltpu.sync_copy(aggr_sh.at[pl.ds(sid * ZR, ZR)],
                        out_hbm.at[cid, pl.ds(sid * ZR, ZR)])

        @pl.when(sid == ns - 1)
        def _():
            pltpu.sync_copy(aggr_sh.at[pl.ds(ns * ZR, tail)],
                            out_hbm.at[cid, pl.ds(ns * ZR, tail)])

    return k(x, src, dst, attr, emb)


def _tc_mlp(x, parts, eps_arr, W1, b1, gamma, beta, W2, b2, *, n, d, blk):
    nb = n // blk
    inv_n = 1.0 / n

    def kf(eps_ref, x_ref, parts_ref, w1_ref, b1_ref, g_ref, be_ref,
           w2_ref, b2_ref, o_ref, h1_scr, sums_scr):
        p = pl.program_id(0)
        i = pl.program_id(1)

        @pl.when(p == 0)
        def _():
            z = ((1.0 + eps_ref[0]) * x_ref[...]
                 + parts_ref[0] + parts_ref[1])
            h = jnp.dot(z, w1_ref[...], preferred_element_type=jnp.float32)
            h = h + b1_ref[...]
            h1_scr[pl.ds(i * blk, blk), :] = h

            @pl.when(i == 0)
            def _():
                sums_scr[...] = jnp.zeros_like(sums_scr)
            sums_scr[0:1, :] += jnp.sum(h, axis=0, keepdims=True)
            sums_scr[1:2, :] += jnp.sum(h * h, axis=0, keepdims=True)

        @pl.when(p == 1)
        def _():
            mu = sums_scr[0:1, :] * inv_n
            var = sums_scr[1:2, :] * inv_n - mu * mu
            a = g_ref[...] * lax.rsqrt(var + BN_EPS)
            c = be_ref[...] - mu * a
            h = jnp.maximum(h1_scr[pl.ds(i * blk, blk), :] * a + c, 0.0)
            o_ref[...] = (jnp.dot(h, w2_ref[...],
                                  preferred_element_type=jnp.float32)
                          + b2_ref[...])

    return pl.pallas_call(
        kf,
        grid=(2, nb),
        in_specs=[
            pl.BlockSpec(memory_space=pltpu.SMEM),
            pl.BlockSpec((blk, d), lambda p, i: ((1 - p) * i, 0)),
            pl.BlockSpec((2, blk, d), lambda p, i: (0, (1 - p) * i, 0)),
            pl.BlockSpec((d, d), lambda p, i: (0, 0)),
            pl.BlockSpec((1, d), lambda p, i: (0, 0)),
            pl.BlockSpec((1, d), lambda p, i: (0, 0)),
            pl.BlockSpec((1, d), lambda p, i: (0, 0)),
            pl.BlockSpec((d, d), lambda p, i: (0, 0)),
            pl.BlockSpec((1, d), lambda p, i: (0, 0)),
        ],
        out_specs=pl.BlockSpec((blk, d), lambda p, i: (i, 0)),
        out_shape=jax.ShapeDtypeStruct((n, d), jnp.float32),
        scratch_shapes=[
            pltpu.VMEM((n, d), jnp.float32),
            pltpu.VMEM((8, d), jnp.float32),
        ],
    )(eps_arr, x, parts, W1, b1, gamma, beta, W2, b2)


def kernel(x, edge_index, edge_attr, edge_emb_w, eps, W1, b1, gamma, beta,
           W2, b2):
    n, d = x.shape
    e = edge_attr.shape[0]
    src = edge_index[0]
    dst = edge_index[1]

    parts = _sc_aggregate(x, src, dst, edge_attr, edge_emb_w, n=n, e=e, d=d)

    blk = 2000
    eps_arr = jnp.reshape(eps, (1,)).astype(jnp.float32)
    out = _tc_mlp(x, parts, eps_arr, W1, jnp.reshape(b1, (1, d)),
                  jnp.reshape(gamma, (1, d)), jnp.reshape(beta, (1, d)),
                  W2, jnp.reshape(b2, (1, d)), n=n, d=d, blk=blk)
    return out
